# baseline (speedup 1.0000x reference)
import functools

import jax
import jax.numpy as jnp
from jax import lax
from jax.experimental import pallas as pl
from jax.experimental.pallas import tpu as pltpu
from jax.experimental.pallas import tpu_sc as plsc

B, S, D = 4, 2048, 1024
NUM_CORES = 2
NUM_SUBCORES = 16
NW = NUM_CORES * NUM_SUBCORES
POS_PER_W = S // NW
R = 16
NBLK = POS_PER_W // R
BLK = R * D


def _build(interpret=False):
  mesh = plsc.VectorSubcoreMesh(
      core_axis_name="c", subcore_axis_name="s",
      num_cores=NUM_CORES, num_subcores=NUM_SUBCORES)

  @functools.partial(
      pl.kernel,
      out_type=jax.ShapeDtypeStruct((B * S * D,), jnp.float32),
      mesh=mesh,
      scratch_types=[
          pltpu.VMEM((BLK,), jnp.float32),
          pltpu.VMEM((BLK,), jnp.float32),
      ],
      interpret=interpret,
  )
  def sc_add(x_hbm, lut_hbm, out_hbm, xbuf, lbuf):
    w = lax.axis_index("s") * NUM_CORES + lax.axis_index("c")
    pos0 = w * POS_PER_W

    for blk in range(NBLK):
      lut_off = (pos0 + blk * R) * D
      pltpu.sync_copy(lut_hbm.at[pl.ds(lut_off, BLK)], lbuf)
      for b in range(B):
        base = b * S * D + lut_off
        pltpu.sync_copy(x_hbm.at[pl.ds(base, BLK)], xbuf)

        @plsc.parallel_loop(0, BLK, 16, unroll=8)
        def add_body(i):
          xbuf[pl.ds(i, 16)] = xbuf[pl.ds(i, 16)] + lbuf[pl.ds(i, 16)]

        pltpu.sync_copy(xbuf, out_hbm.at[pl.ds(base, BLK)])

  return sc_add


_sc_add = _build()


@jax.jit
def kernel(x, lut):
  out = _sc_add(x.reshape(-1), lut.reshape(-1))
  return out.reshape(B, S, D)


# speedup vs baseline: 1.2085x; 1.2085x over previous
import functools

import jax
import jax.numpy as jnp
from jax import lax
from jax.experimental import pallas as pl
from jax.experimental.pallas import tpu as pltpu
from jax.experimental.pallas import tpu_sc as plsc

B, S, D = 4, 2048, 1024
NUM_CORES = 2
NUM_SUBCORES = 16
NW = NUM_CORES * NUM_SUBCORES
POS_PER_W = S // NW
R = 16
NLB = POS_PER_W // R
NSTEP = NLB * B
BLK = R * D
NR = 4
L = 3


def _build(interpret=False):
  mesh = plsc.VectorSubcoreMesh(
      core_axis_name="c", subcore_axis_name="s",
      num_cores=NUM_CORES, num_subcores=NUM_SUBCORES)

  scratch = (
      [pltpu.VMEM((BLK,), jnp.float32) for _ in range(NR)]
      + [pltpu.VMEM((BLK,), jnp.float32) for _ in range(2)]
      + [pltpu.SemaphoreType.DMA for _ in range(2 * NR + 2)]
  )

  @functools.partial(
      pl.kernel,
      out_type=jax.ShapeDtypeStruct((B * S * D,), jnp.float32),
      mesh=mesh,
      scratch_types=scratch,
      interpret=interpret,
  )
  def sc_add(x_hbm, lut_hbm, out_hbm, *scr):
    xbuf = scr[:NR]
    lbuf = scr[NR:NR + 2]
    sem_ld = scr[NR + 2:NR + 2 + NR]
    sem_st = scr[NR + 2 + NR:NR + 2 + 2 * NR]
    sem_lut = scr[NR + 2 + 2 * NR:]

    w = lax.axis_index("s") * NUM_CORES + lax.axis_index("c")
    pos0 = w * POS_PER_W

    def xoff(step):
      lb, b = divmod(step, B)
      return (b * S + pos0 + lb * R) * D

    loads, lloads, stores = {}, {}, {}
    waited = set()

    def issue_load(s):
      r = s % NR
      loads[s] = pltpu.async_copy(
          x_hbm.at[pl.ds(xoff(s), BLK)], xbuf[r], sem_ld[r])

    def issue_lut(lb):
      lloads[lb] = pltpu.async_copy(
          lut_hbm.at[pl.ds((pos0 + lb * R) * D, BLK)],
          lbuf[lb % 2], sem_lut[lb % 2])

    issue_lut(0)
    if NLB > 1:
      issue_lut(1)
    for s in range(min(L, NSTEP)):
      issue_load(s)

    for s in range(NSTEP):
      lb, b = divmod(s, B)
      ss = s + L
      if ss < NSTEP:
        if ss - NR >= 0:
          stores[ss - NR].wait()
          waited.add(ss - NR)
        issue_load(ss)
      if b == 0:
        lloads[lb].wait()
      r = s % NR
      loads[s].wait()
      xb, lbf = xbuf[r], lbuf[lb % 2]

      @plsc.parallel_loop(0, BLK, 16, unroll=8)
      def add_body(i):
        xb[pl.ds(i, 16)] = xb[pl.ds(i, 16)] + lbf[pl.ds(i, 16)]

      stores[s] = pltpu.async_copy(
          xb, out_hbm.at[pl.ds(xoff(s), BLK)], sem_st[r])
      if b == B - 1 and lb + 2 < NLB:
        issue_lut(lb + 2)

    for s in range(NSTEP):
      if s not in waited:
        stores[s].wait()

  return sc_add


_sc_add = _build()


@jax.jit
def kernel(x, lut):
  out = _sc_add(x.reshape(-1), lut.reshape(-1))
  return out.reshape(B, S, D)


# speedup vs baseline: 2.9104x; 2.4082x over previous
import functools

import jax
import jax.numpy as jnp
from jax import lax
from jax.experimental import pallas as pl
from jax.experimental.pallas import tpu as pltpu
from jax.experimental.pallas import tpu_sc as plsc

B, S, D = 4, 2048, 1024
NUM_CORES = 2
NUM_SUBCORES = 16
NW = NUM_CORES * NUM_SUBCORES
POS_PER_W = S // NW
R = 16
NLB = POS_PER_W // R
NSTEP = NLB * B
NR = 4
L = 3


def _build(interpret=False):
  mesh = plsc.VectorSubcoreMesh(
      core_axis_name="c", subcore_axis_name="s",
      num_cores=NUM_CORES, num_subcores=NUM_SUBCORES)

  scratch = (
      [pltpu.VMEM((R, D), jnp.float32) for _ in range(NR)]
      + [pltpu.VMEM((R, D), jnp.float32) for _ in range(2)]
      + [pltpu.SemaphoreType.DMA for _ in range(2 * NR + 2)]
  )

  @functools.partial(
      pl.kernel,
      out_type=jax.ShapeDtypeStruct((B, S, D), jnp.float32),
      mesh=mesh,
      scratch_types=scratch,
      interpret=interpret,
  )
  def sc_add(x_hbm, lut_hbm, out_hbm, *scr):
    xbuf = scr[:NR]
    lbuf = scr[NR:NR + 2]
    sem_ld = scr[NR + 2:NR + 2 + NR]
    sem_st = scr[NR + 2 + NR:NR + 2 + 2 * NR]
    sem_lut = scr[NR + 2 + 2 * NR:]

    w = lax.axis_index("s") * NUM_CORES + lax.axis_index("c")
    pos0 = w * POS_PER_W

    loads, lloads, stores = {}, {}, {}
    waited = set()

    def issue_load(s):
      lb, b = divmod(s, B)
      r = s % NR
      loads[s] = pltpu.async_copy(
          x_hbm.at[b, pl.ds(pos0 + lb * R, R), :], xbuf[r], sem_ld[r])

    def issue_lut(lb):
      lloads[lb] = pltpu.async_copy(
          lut_hbm.at[pl.ds(pos0 + lb * R, R), :], lbuf[lb % 2],
          sem_lut[lb % 2])

    issue_lut(0)
    if NLB > 1:
      issue_lut(1)
    for s in range(min(L, NSTEP)):
      issue_load(s)

    for s in range(NSTEP):
      lb, b = divmod(s, B)
      ss = s + L
      if ss < NSTEP:
        if ss - NR >= 0:
          stores[ss - NR].wait()
          waited.add(ss - NR)
        issue_load(ss)
      if b == 0:
        lloads[lb].wait()
      r = s % NR
      loads[s].wait()
      xb, lbf = xbuf[r], lbuf[lb % 2]

      @plsc.parallel_loop(0, R * D, 16, unroll=8)
      def add_body(i):
        row = i >> 10
        col = pl.multiple_of(i & (D - 1), 16)
        xb[row, pl.ds(col, 16)] = (
            xb[row, pl.ds(col, 16)] + lbf[row, pl.ds(col, 16)])

      stores[s] = pltpu.async_copy(
          xb, out_hbm.at[b, pl.ds(pos0 + lb * R, R), :], sem_st[r])
      if b == B - 1 and lb + 2 < NLB:
        issue_lut(lb + 2)

    for s in range(NSTEP):
      if s not in waited:
        stores[s].wait()

  return sc_add


_sc_add = _build()


@jax.jit
def kernel(x, lut):
  return _sc_add(x, lut)


# speedup vs baseline: 2.9249x; 1.0050x over previous
import functools

import jax
import jax.numpy as jnp
from jax import lax
from jax.experimental import pallas as pl
from jax.experimental.pallas import tpu as pltpu
from jax.experimental.pallas import tpu_sc as plsc

B, S, D = 4, 2048, 1024
NUM_CORES = 2
NUM_SUBCORES = 16
NW = NUM_CORES * NUM_SUBCORES
POS_PER_W = S // NW
R = 16
NLB = POS_PER_W // R
NSTEP = NLB * B
NR = 4
L = 3


def _build(interpret=False):
  mesh = plsc.VectorSubcoreMesh(
      core_axis_name="c", subcore_axis_name="s",
      num_cores=NUM_CORES, num_subcores=NUM_SUBCORES)

  scratch = (
      [pltpu.VMEM((R, D), jnp.float32) for _ in range(NR)]
      + [pltpu.VMEM((R, D), jnp.float32) for _ in range(2)]
      + [pltpu.SemaphoreType.DMA for _ in range(2 * NR + 2)]
  )

  @functools.partial(
      pl.kernel,
      out_type=jax.ShapeDtypeStruct((B, S, D), jnp.float32),
      mesh=mesh,
      scratch_types=scratch,
      interpret=interpret,
  )
  def sc_add(x_hbm, lut_hbm, out_hbm, *scr):
    xbuf = scr[:NR]
    lbuf = scr[NR:NR + 2]
    sem_ld = scr[NR + 2:NR + 2 + NR]
    sem_st = scr[NR + 2 + NR:NR + 2 + 2 * NR]
    sem_lut = scr[NR + 2 + 2 * NR:]

    w = lax.axis_index("s") * NUM_CORES + lax.axis_index("c")
    pos0 = w * POS_PER_W

    loads, lloads, stores = {}, {}, {}
    waited = set()

    def issue_load(s):
      lb, b = divmod(s, B)
      r = s % NR
      loads[s] = pltpu.async_copy(
          x_hbm.at[b, pl.ds(pos0 + lb * R, R), :], xbuf[r], sem_ld[r])

    def issue_lut(lb):
      lloads[lb] = pltpu.async_copy(
          lut_hbm.at[pl.ds(pos0 + lb * R, R), :], lbuf[lb % 2],
          sem_lut[lb % 2])

    issue_lut(0)
    if NLB > 1:
      issue_lut(1)
    for s in range(min(L, NSTEP)):
      issue_load(s)

    for s in range(NSTEP):
      lb, b = divmod(s, B)
      ss = s + L
      if ss < NSTEP:
        if ss - NR >= 0:
          stores[ss - NR].wait()
          waited.add(ss - NR)
        issue_load(ss)
      if b == 0:
        lloads[lb].wait()
      r = s % NR
      loads[s].wait()
      xb, lbf = xbuf[r], lbuf[lb % 2]

      @plsc.parallel_loop(0, R * D, 16, unroll=8)
      def add_body(i):
        row = i >> 10
        col = pl.multiple_of(i & (D - 1), 16)
        plsc.addupdate(xb.at[row, pl.ds(col, 16)], lbf[row, pl.ds(col, 16)])

      stores[s] = pltpu.async_copy(
          xb, out_hbm.at[b, pl.ds(pos0 + lb * R, R), :], sem_st[r])
      if b == B - 1 and lb + 2 < NLB:
        issue_lut(lb + 2)

    for s in range(NSTEP):
      if s not in waited:
        stores[s].wait()

  return sc_add


_sc_add = _build()


@jax.jit
def kernel(x, lut):
  return _sc_add(x, lut)
